# baseline (speedup 1.0000x reference)
import functools

import numpy as np

import jax
import jax.numpy as jnp
from jax import lax
from jax.experimental import pallas as pl
from jax.experimental.pallas import tpu as pltpu
from jax.experimental.pallas import tpu_sc as plsc

_NCORES = 2
_NSUB = 16


def _lrelu(x):
    return jnp.where(x > 0, x, 0.01 * x)


def _k1_body(R, feat_ref, w_ref, rootpad_ref, h_ref, tab_ref):
    i = pl.program_id(0)
    blk = feat_ref[...]
    h_ref[...] = jnp.dot(blk, w_ref[...], preferred_element_type=jnp.float32)
    rid = lax.broadcasted_iota(jnp.int32, (R, 1), 0) + i * R
    ohr = (rid == rootpad_ref[...]).astype(jnp.float32)
    part = lax.dot_general(ohr, blk, (((0,), (0,)), ((), ())),
                           preferred_element_type=jnp.float32)

    @pl.when(i == 0)
    def _():
        tab_ref[...] = jnp.zeros_like(tab_ref)

    tab_ref[...] += part


def _k1(features, W1, rootpad, R):
    N, D = features.shape
    H = W1.shape[1]
    nb = N // R
    return pl.pallas_call(
        functools.partial(_k1_body, R),
        grid=(nb,),
        in_specs=[
            pl.BlockSpec((R, D), lambda i: (i, 0)),
            pl.BlockSpec((D, H), lambda i: (0, 0)),
            pl.BlockSpec((1, 128), lambda i: (0, 0)),
        ],
        out_specs=[
            pl.BlockSpec((R, H), lambda i: (i, 0)),
            pl.BlockSpec((128, D), lambda i: (0, 0)),
        ],
        out_shape=[
            jax.ShapeDtypeStruct((N, H), jnp.float32),
            jax.ShapeDtypeStruct((128, D), jnp.float32),
        ],
    )(features, W1, rootpad)


def _stage_body(R, inner_lrelu, want_table, want_post, refs):
    if want_table:
        (p_ref, bpre_ref, wa_ref, tab_ref, wb_ref, batch_ref,
         rootpad_ref, out_ref, tabout_ref) = refs
    else:
        (p_ref, bpre_ref, wa_ref, tab_ref, wb_ref, batch_ref,
         bpost_ref, out_ref) = refs
    i = pl.program_id(0)
    xp = p_ref[...] + bpre_ref[...]
    x = _lrelu(xp)
    t = tab_ref[...]
    if inner_lrelu:
        t = _lrelu(t)
    tt = jnp.dot(t, wb_ref[...], preferred_element_type=jnp.float32)
    ohb = (batch_ref[...] == lax.broadcasted_iota(jnp.int32, (1, 128), 1))
    ohb = ohb.astype(jnp.float32)
    acc = jnp.dot(x, wa_ref[...], preferred_element_type=jnp.float32)
    acc = acc + jnp.dot(ohb, tt, preferred_element_type=jnp.float32)
    if want_post:
        acc = _lrelu(acc + bpost_ref[...])
    out_ref[...] = acc.astype(out_ref.dtype)
    if want_table:
        rid = lax.broadcasted_iota(jnp.int32, (R, 1), 0) + i * R
        ohr = (rid == rootpad_ref[...]).astype(jnp.float32)
        part = lax.dot_general(ohr, xp, (((0,), (0,)), ((), ())),
                               preferred_element_type=jnp.float32)

        @pl.when(i == 0)
        def _():
            tabout_ref[...] = jnp.zeros_like(tabout_ref)

        tabout_ref[...] += part


def _stage(p, bpre, wa, tab, wb, batch_col, rootpad, bpost, R,
           inner_lrelu, want_table, want_post):
    N, D = p.shape
    H = wa.shape[1]
    nb = N // R
    full = lambda shape: pl.BlockSpec(shape, lambda i: tuple(0 for _ in shape))
    blk = pl.BlockSpec((R, D), lambda i: (i, 0))
    in_specs = [blk, full((1, D)), full((D, H)), full((128, D)),
                full((D, H)), pl.BlockSpec((R, 1), lambda i: (i, 0))]
    args = [p, bpre, wa, tab, wb, batch_col]
    if want_table:
        in_specs.append(full((1, 128)))
        args.append(rootpad)
    if want_post:
        in_specs.append(full((1, H)))
        args.append(bpost)
    out_specs = [pl.BlockSpec((R, H), lambda i: (i, 0))]
    out_shape = [jax.ShapeDtypeStruct((N, H), jnp.float32)]
    if want_table:
        out_specs.append(full((128, D)))
        out_shape.append(jax.ShapeDtypeStruct((128, D), jnp.float32))
    body = functools.partial(_stage_body, R, inner_lrelu, want_table, want_post)
    res = pl.pallas_call(
        lambda *refs: body(refs),
        grid=(nb,),
        in_specs=in_specs,
        out_specs=out_specs,
        out_shape=out_shape,
    )(*args)
    return res if want_table else (res[0], None)


def _make_sc_agg(N, D, nch, C):
    mesh = plsc.VectorSubcoreMesh(core_axis_name="c", subcore_axis_name="s",
                                  num_cores=_NCORES, num_subcores=_NSUB)
    per = (-(-N // _NSUB) + 7) // 8 * 8
    last = N - (_NSUB - 1) * per
    assert last > 0 and last % 8 == 0 and per % 8 == 0
    assert nch % 4 == 0
    grp = D // 16

    def _range_chunks(length):
        out = [(j * C, C) for j in range(length // C)]
        if length % C:
            out.append((length // C * C, length % C))
        return out

    @functools.partial(
        pl.kernel,
        out_type=jax.ShapeDtypeStruct((N, D), jnp.float32),
        mesh=mesh,
        scratch_types=[
            pltpu.VMEM((4, 2, C), jnp.int32),
            pltpu.VMEM((4, C), jnp.float32),
            pltpu.VMEM((2, C), jnp.int32),
            pltpu.VMEM((4, C, D), jnp.float32),
            pltpu.VMEM_SHARED((N, D), jnp.float32),
        ] + [pltpu.SemaphoreType.DMA] * 13,
    )
    def agg(h_hbm, comb_hbm, val_hbm, out_hbm,
            comb, vv, didx, rows, acc,
            g0, g1, g2, g3, s0, s1, s2, s3, i0, i1, i2, i3, wsem):
        c_ax = lax.axis_index("c")
        s_ax = lax.axis_index("s")
        gsems = (g0, g1, g2, g3)
        ssems = (s0, s1, s2, s3)
        isems = (i0, i1, i2, i3)

        @pl.when(c_ax == 0)
        def _core0():
            gbase = s_ax * nch
            base = s_ax * per

            zero = jnp.zeros((16,), jnp.float32)
            for r in range(C):
                for k in range(grp):
                    rows[0, r, pl.ds(k * 16, 16)] = zero

            def _sweep(to_spmem):
                def _do(length):
                    def _inner():
                        chunks = _range_chunks(length)
                        for start, cnt in chunks:
                            a = acc.at[pl.ds(base + start, cnt)]
                            if to_spmem:
                                pltpu.async_copy(rows.at[0, pl.ds(0, cnt)],
                                                 a, wsem)
                            else:
                                pltpu.async_copy(
                                    a, out_hbm.at[pl.ds(base + start, cnt)],
                                    wsem)
                        for start, cnt in chunks:
                            a = acc.at[pl.ds(base + start, cnt)]
                            if to_spmem:
                                pltpu.make_async_copy(
                                    rows.at[0, pl.ds(0, cnt)], a, wsem).wait()
                            else:
                                pltpu.make_async_copy(
                                    a, out_hbm.at[pl.ds(base + start, cnt)],
                                    wsem).wait()
                    return _inner
                pl.when(s_ax < _NSUB - 1)(_do(per))
                pl.when(s_ax == _NSUB - 1)(_do(last))

            _sweep(to_spmem=True)
            plsc.subcore_barrier()

            def issue_comb(chunk, slot):
                pltpu.async_copy(comb_hbm.at[chunk], comb.at[slot],
                                 isems[slot])
                pltpu.async_copy(val_hbm.at[chunk], vv.at[slot], isems[slot])

            def wait_comb(slot):
                pltpu.make_async_copy(comb_hbm.at[0], comb.at[0],
                                      isems[slot]).wait()
                pltpu.make_async_copy(val_hbm.at[0], vv.at[0],
                                      isems[slot]).wait()

            def issue_gather(slot, chunk_slot):
                h2_ = C // 2
                pltpu.async_copy(
                    h_hbm.at[comb.at[chunk_slot, 0, pl.ds(0, h2_)]],
                    rows.at[slot, pl.ds(0, h2_)], gsems[slot])
                pltpu.async_copy(
                    h_hbm.at[comb.at[chunk_slot, 0, pl.ds(h2_, h2_)]],
                    rows.at[slot, pl.ds(h2_, h2_)], gsems[slot])

            def drain_gather(slot):
                for _ in range(2):
                    pltpu.make_async_copy(h_hbm.at[pl.ds(0, C // 2)],
                                          rows.at[slot, pl.ds(0, C // 2)],
                                          gsems[slot]).wait()

            def drain_scatter(slot):
                pltpu.make_async_copy(h_hbm.at[pl.ds(0, C)], rows.at[slot],
                                      ssems[slot]).wait()

            for p_ in range(4):
                issue_comb(gbase + min(p_, nch - 1), p_)
            for b in range(2):
                wait_comb(b)
                issue_gather(b, b)

            @pl.loop(0, nch, step=4)
            def _(i):
                for b in range(4):
                    j = i + b
                    nb_ = (b + 2) % 4
                    drain_gather(b)
                    pl.when(j >= 2)(lambda: drain_scatter(nb_))
                    wait_comb(nb_)
                    issue_gather(nb_, nb_)
                    for g in range(C // 16):
                        sl = pl.ds(g * 16, 16)
                        didx[b % 2, sl] = comb[b, 1, sl]

                    @pl.loop(0, C // 16)
                    def _(g):
                        v16 = vv[b, pl.ds(g * 16, 16)]
                        for jj in range(16):
                            splat = v16.at[jnp.full((16,), jj,
                                                    jnp.int32)].get(
                                mode="promise_in_bounds")
                            r = g * 16 + jj
                            for k in range(grp):
                                sl = pl.ds(k * 16, 16)
                                rows[b, r, sl] = rows[b, r, sl] * splat

                    pltpu.async_copy(rows.at[b], acc.at[didx.at[b % 2]],
                                     ssems[b], add=True)
                    issue_comb(gbase + jnp.minimum(j + 4, nch - 1), b)

            for b in (0, 1):
                drain_gather(b)
            for b in (2, 3):
                drain_scatter(b)
                wait_comb(b)
            plsc.subcore_barrier()
            _sweep(to_spmem=False)

    return agg


def kernel(features, adjs, values, root_idx, propagation_node_num,
           propagation_edge_num, batch, W1, b1, W2, b2, Wlin, blin):
    N, IN = features.shape
    E = adjs.shape[1]
    HID = W1.shape[1]
    OUT = W2.shape[1]
    B = root_idx.shape[0]
    C = 96
    nch = (-(-E // (_NSUB * C)) + 3) // 4 * 4
    tot = _NSUB * nch
    R = 1000

    pad = tot * C - E
    zpad_i = jnp.zeros((pad,), jnp.int32)
    src_r = jnp.concatenate([adjs[0], zpad_i]).reshape(tot, 1, C)
    dst_r = jnp.concatenate([adjs[1], zpad_i]).reshape(tot, 1, C)
    comb_r = jnp.concatenate([src_r, dst_r], axis=1)
    val_r = jnp.concatenate(
        [values, jnp.zeros((pad,), jnp.float32)]).reshape(tot, C)
    rootpad = jnp.concatenate(
        [root_idx.astype(jnp.int32),
         jnp.full((128 - B,), -1, jnp.int32)]).reshape(1, 128)
    batch_col = batch.astype(jnp.int32).reshape(N, 1)
    b1r = b1.reshape(1, HID)
    b2r = b2.reshape(1, OUT)
    blinr = blin.reshape(1, IN)
    W2a = W2[:HID]
    W2b = W2[HID:]
    WlinA = Wlin[:OUT]
    WlinB = Wlin[OUT:]
    b1p = b1.reshape(1, HID)
    b2p = b2.reshape(1, OUT)

    agg = _make_sc_agg(N, HID, nch, C)

    h1, root_tab = _k1(features, W1, rootpad, R)
    p = agg(h1, comb_r, val_r)
    h2, f1_tab = _stage(p, b1p, W2a, root_tab, W2b, batch_col,
                        rootpad, None, R, inner_lrelu=True, want_table=True,
                        want_post=False)
    q = agg(h2, comb_r, val_r)
    out, _ = _stage(q, b2p, WlinA, f1_tab, WlinB, batch_col,
                    None, blinr, R, inner_lrelu=False, want_table=False,
                    want_post=True)
    return out


# speedup vs baseline: 1.3877x; 1.3877x over previous
import functools

import numpy as np

import jax
import jax.numpy as jnp
from jax import lax
from jax.experimental import pallas as pl
from jax.experimental.pallas import tpu as pltpu
from jax.experimental.pallas import tpu_sc as plsc

_NCORES = 2
_NSUB = 16


def _lrelu(x):
    return jnp.where(x > 0, x, 0.01 * x)


def _k1_body(R, feat_ref, w_ref, rootpad_ref, h_ref, tab_ref):
    i = pl.program_id(0)
    blk = feat_ref[...]
    h_ref[...] = jnp.dot(blk, w_ref[...], preferred_element_type=jnp.float32)
    rid = lax.broadcasted_iota(jnp.int32, (R, 1), 0) + i * R
    ohr = (rid == rootpad_ref[...]).astype(jnp.float32)
    part = lax.dot_general(ohr, blk, (((0,), (0,)), ((), ())),
                           preferred_element_type=jnp.float32)

    @pl.when(i == 0)
    def _():
        tab_ref[...] = jnp.zeros_like(tab_ref)

    tab_ref[...] += part


def _k1(features, W1, rootpad, R):
    N, D = features.shape
    H = W1.shape[1]
    nb = N // R
    return pl.pallas_call(
        functools.partial(_k1_body, R),
        grid=(nb,),
        in_specs=[
            pl.BlockSpec((R, D), lambda i: (i, 0)),
            pl.BlockSpec((D, H), lambda i: (0, 0)),
            pl.BlockSpec((1, 128), lambda i: (0, 0)),
        ],
        out_specs=[
            pl.BlockSpec((R, H), lambda i: (i, 0)),
            pl.BlockSpec((128, D), lambda i: (0, 0)),
        ],
        out_shape=[
            jax.ShapeDtypeStruct((N, H), jnp.float32),
            jax.ShapeDtypeStruct((128, D), jnp.float32),
        ],
    )(features, W1, rootpad)


def _stage_body(R, inner_lrelu, want_table, want_post, refs):
    if want_table:
        (p_ref, bpre_ref, wa_ref, tab_ref, wb_ref, batch_ref,
         rootpad_ref, out_ref, tabout_ref) = refs
    else:
        (p_ref, bpre_ref, wa_ref, tab_ref, wb_ref, batch_ref,
         bpost_ref, out_ref) = refs
    i = pl.program_id(0)
    xp = p_ref[...] + bpre_ref[...]
    x = _lrelu(xp)
    t = tab_ref[...]
    if inner_lrelu:
        t = _lrelu(t)
    tt = jnp.dot(t, wb_ref[...], preferred_element_type=jnp.float32)
    ohb = (batch_ref[...] == lax.broadcasted_iota(jnp.int32, (1, 128), 1))
    ohb = ohb.astype(jnp.float32)
    acc = jnp.dot(x, wa_ref[...], preferred_element_type=jnp.float32)
    acc = acc + jnp.dot(ohb, tt, preferred_element_type=jnp.float32)
    if want_post:
        acc = _lrelu(acc + bpost_ref[...])
    out_ref[...] = acc.astype(out_ref.dtype)
    if want_table:
        rid = lax.broadcasted_iota(jnp.int32, (R, 1), 0) + i * R
        ohr = (rid == rootpad_ref[...]).astype(jnp.float32)
        part = lax.dot_general(ohr, xp, (((0,), (0,)), ((), ())),
                               preferred_element_type=jnp.float32)

        @pl.when(i == 0)
        def _():
            tabout_ref[...] = jnp.zeros_like(tabout_ref)

        tabout_ref[...] += part


def _stage(p, bpre, wa, tab, wb, batch_col, rootpad, bpost, R,
           inner_lrelu, want_table, want_post):
    N, D = p.shape
    H = wa.shape[1]
    nb = N // R
    full = lambda shape: pl.BlockSpec(shape, lambda i: tuple(0 for _ in shape))
    blk = pl.BlockSpec((R, D), lambda i: (i, 0))
    in_specs = [blk, full((1, D)), full((D, H)), full((128, D)),
                full((D, H)), pl.BlockSpec((R, 1), lambda i: (i, 0))]
    args = [p, bpre, wa, tab, wb, batch_col]
    if want_table:
        in_specs.append(full((1, 128)))
        args.append(rootpad)
    if want_post:
        in_specs.append(full((1, H)))
        args.append(bpost)
    out_specs = [pl.BlockSpec((R, H), lambda i: (i, 0))]
    out_shape = [jax.ShapeDtypeStruct((N, H), jnp.float32)]
    if want_table:
        out_specs.append(full((128, D)))
        out_shape.append(jax.ShapeDtypeStruct((128, D), jnp.float32))
    body = functools.partial(_stage_body, R, inner_lrelu, want_table, want_post)
    res = pl.pallas_call(
        lambda *refs: body(refs),
        grid=(nb,),
        in_specs=in_specs,
        out_specs=out_specs,
        out_shape=out_shape,
    )(*args)
    return res if want_table else (res[0], None)


def _make_sc_agg(N, D, nch, C):
    mesh = plsc.VectorSubcoreMesh(core_axis_name="c", subcore_axis_name="s",
                                  num_cores=_NCORES, num_subcores=_NSUB)
    per = (-(-N // _NSUB) + 7) // 8 * 8
    last = N - (_NSUB - 1) * per
    assert last > 0 and last % 8 == 0 and per % 8 == 0
    assert nch % 4 == 0
    grp = D // 16

    def _range_chunks(length):
        out = [(j * C, C) for j in range(length // C)]
        if length % C:
            out.append((length // C * C, length % C))
        return out

    @functools.partial(
        pl.kernel,
        out_type=jax.ShapeDtypeStruct((N, D), jnp.float32),
        mesh=mesh,
        scratch_types=[
            pltpu.VMEM((4, 2, C), jnp.int32),
            pltpu.VMEM((4, C), jnp.float32),
            pltpu.VMEM((2, C), jnp.int32),
            pltpu.VMEM((4, C, D), jnp.float32),
            pltpu.VMEM_SHARED((N, D), jnp.float32),
        ] + [pltpu.SemaphoreType.DMA] * 13,
    )
    def agg(h_hbm, comb_hbm, val_hbm, out_hbm,
            comb, vv, didx, rows, acc,
            g0, g1, g2, g3, s0, s1, s2, s3, i0, i1, i2, i3, wsem):
        c_ax = lax.axis_index("c")
        s_ax = lax.axis_index("s")
        gsems = (g0, g1, g2, g3)
        ssems = (s0, s1, s2, s3)
        isems = (i0, i1, i2, i3)

        @pl.when(c_ax == 0)
        def _core0():
            gbase = s_ax * nch
            base = s_ax * per

            zero = jnp.zeros((16,), jnp.float32)
            for r in range(C):
                for k in range(grp):
                    rows[0, r, pl.ds(k * 16, 16)] = zero

            def _sweep(to_spmem):
                def _do(length):
                    def _inner():
                        chunks = _range_chunks(length)
                        for start, cnt in chunks:
                            a = acc.at[pl.ds(base + start, cnt)]
                            if to_spmem:
                                pltpu.async_copy(rows.at[0, pl.ds(0, cnt)],
                                                 a, wsem)
                            else:
                                pltpu.async_copy(
                                    a, out_hbm.at[pl.ds(base + start, cnt)],
                                    wsem)
                        for start, cnt in chunks:
                            a = acc.at[pl.ds(base + start, cnt)]
                            if to_spmem:
                                pltpu.make_async_copy(
                                    rows.at[0, pl.ds(0, cnt)], a, wsem).wait()
                            else:
                                pltpu.make_async_copy(
                                    a, out_hbm.at[pl.ds(base + start, cnt)],
                                    wsem).wait()
                    return _inner
                pl.when(s_ax < _NSUB - 1)(_do(per))
                pl.when(s_ax == _NSUB - 1)(_do(last))

            _sweep(to_spmem=True)
            plsc.subcore_barrier()

            def issue_comb(chunk, slot):
                pltpu.async_copy(comb_hbm.at[chunk], comb.at[slot],
                                 isems[slot])
                pltpu.async_copy(val_hbm.at[chunk], vv.at[slot], isems[slot])

            def wait_comb(slot):
                pltpu.make_async_copy(comb_hbm.at[0], comb.at[0],
                                      isems[slot]).wait()
                pltpu.make_async_copy(val_hbm.at[0], vv.at[0],
                                      isems[slot]).wait()

            def issue_gather(slot, chunk_slot):
                h2_ = C // 2
                pltpu.async_copy(
                    h_hbm.at[comb.at[chunk_slot, 0, pl.ds(0, h2_)]],
                    rows.at[slot, pl.ds(0, h2_)], gsems[slot])
                pltpu.async_copy(
                    h_hbm.at[comb.at[chunk_slot, 0, pl.ds(h2_, h2_)]],
                    rows.at[slot, pl.ds(h2_, h2_)], gsems[slot])

            def drain_gather(slot):
                for _ in range(2):
                    pltpu.make_async_copy(h_hbm.at[pl.ds(0, C // 2)],
                                          rows.at[slot, pl.ds(0, C // 2)],
                                          gsems[slot]).wait()

            def drain_scatter(slot):
                pltpu.make_async_copy(h_hbm.at[pl.ds(0, C)], rows.at[slot],
                                      ssems[slot]).wait()

            for p_ in range(4):
                issue_comb(gbase + min(p_, nch - 1), p_)
            for b in range(2):
                wait_comb(b)
                issue_gather(b, b)

            @pl.loop(0, nch, step=4)
            def _(i):
                for b in range(4):
                    j = i + b
                    nb_ = (b + 2) % 4
                    drain_gather(b)
                    pl.when(j >= 2)(lambda: drain_scatter(nb_))
                    wait_comb(nb_)
                    issue_gather(nb_, nb_)
                    for g in range(C // 16):
                        sl = pl.ds(g * 16, 16)
                        didx[b % 2, sl] = comb[b, 1, sl]

                    @pl.loop(0, C // 16)
                    def _(g):
                        v16 = vv[b, pl.ds(g * 16, 16)]
                        for jj in range(16):
                            splat = v16.at[jnp.full((16,), jj,
                                                    jnp.int32)].get(
                                mode="promise_in_bounds")
                            r = g * 16 + jj
                            for k in range(grp):
                                sl = pl.ds(k * 16, 16)
                                rows[b, r, sl] = rows[b, r, sl] * splat

                    pltpu.async_copy(rows.at[b], acc.at[didx.at[b % 2]],
                                     ssems[b], add=True)
                    issue_comb(gbase + jnp.minimum(j + 4, nch - 1), b)

            for b in (0, 1):
                drain_gather(b)
            for b in (2, 3):
                drain_scatter(b)
                wait_comb(b)
            plsc.subcore_barrier()
            _sweep(to_spmem=False)

    return agg


def kernel(features, adjs, values, root_idx, propagation_node_num,
           propagation_edge_num, batch, W1, b1, W2, b2, Wlin, blin):
    N, IN = features.shape
    E = adjs.shape[1]
    HID = W1.shape[1]
    OUT = W2.shape[1]
    B = root_idx.shape[0]
    C = 80
    nch = (-(-E // (_NSUB * C)) + 3) // 4 * 4
    tot = _NSUB * nch
    R = 1000

    pad = tot * C - E
    zpad_i = jnp.zeros((pad,), jnp.int32)
    src_r = jnp.concatenate([adjs[0], zpad_i]).reshape(tot, 1, C)
    dst_r = jnp.concatenate([adjs[1], zpad_i]).reshape(tot, 1, C)
    comb_r = jnp.concatenate([src_r, dst_r], axis=1)
    val_r = jnp.concatenate(
        [values, jnp.zeros((pad,), jnp.float32)]).reshape(tot, C)
    rootpad = jnp.concatenate(
        [root_idx.astype(jnp.int32),
         jnp.full((128 - B,), -1, jnp.int32)]).reshape(1, 128)
    batch_col = batch.astype(jnp.int32).reshape(N, 1)
    b1r = b1.reshape(1, HID)
    b2r = b2.reshape(1, OUT)
    blinr = blin.reshape(1, IN)
    W2a = W2[:HID]
    W2b = W2[HID:]
    WlinA = Wlin[:OUT]
    WlinB = Wlin[OUT:]
    b1p = b1.reshape(1, HID)
    b2p = b2.reshape(1, OUT)

    agg = _make_sc_agg(N, HID, nch, C)

    h1, root_tab = _k1(features, W1, rootpad, R)
    p = agg(h1, comb_r, val_r)
    h2, f1_tab = _stage(p, b1p, W2a, root_tab, W2b, batch_col,
                        rootpad, None, R, inner_lrelu=True, want_table=True,
                        want_post=False)
    q = agg(h2, comb_r, val_r)
    out, _ = _stage(q, b2p, WlinA, f1_tab, WlinB, batch_col,
                    None, blinr, R, inner_lrelu=False, want_table=False,
                    want_post=True)
    return out
